# baseline (speedup 1.0000x reference)
import functools

import jax
import jax.numpy as jnp
from jax.experimental import pallas as pl
from jax.experimental.pallas import tpu as pltpu
from jax.experimental.pallas import tpu_sc as plsc

N = 10000
D = 128
H = 128
K = 16
NP = 10240
RA = 512
NA = 10240
RB = 400
RC = 200
GW = 128
EPS = 1e-5


def _knn(xA):
    grid = NA // RA

    def kern(xall_ref, xb_ref, idx_ref, strip_ref):
        xall = xall_ref[...]
        ones = jnp.ones((1, D), jnp.float32)
        sqc = jax.lax.dot_general(
            ones, xall * xall, (((1,), (1,)), ((), ())),
            preferred_element_type=jnp.float32)
        xb = xb_ref[...]
        sqb = jnp.sum(xb * xb, axis=1, keepdims=True)
        dot = jax.lax.dot_general(
            xb, xall, (((1,), (1,)), ((), ())),
            preferred_element_type=jnp.float32)
        cols = jax.lax.broadcasted_iota(jnp.int32, (RA, NP), 1)
        pad = jnp.where(cols >= N, jnp.float32(1e30), jnp.float32(0.0))
        strip_ref[...] = sqb + sqc + pad - 2.0 * dot
        v = jnp.full((RA, 1), -jnp.inf, jnp.float32)
        j = jnp.full((RA, 1), -1, jnp.int32)
        for t in range(K):
            s = strip_ref[...]
            succ = (s > v) | ((s == v) & (cols > j))
            m = jnp.min(jnp.where(succ, s, jnp.float32(jnp.inf)),
                        axis=1, keepdims=True)
            sel = (s == m) & succ
            am = jnp.min(jnp.where(sel, cols, NP), axis=1, keepdims=True)
            idx_ref[t, :] = am[:, 0]
            v, j = m, am

    return pl.pallas_call(
        kern,
        grid=(grid,),
        in_specs=[
            pl.BlockSpec((NP, D), lambda i: (0, 0)),
            pl.BlockSpec((RA, D), lambda i: (i, 0)),
        ],
        out_specs=pl.BlockSpec((K, RA), lambda i: (0, i)),
        out_shape=jax.ShapeDtypeStruct((K, NA), jnp.int32),
        scratch_shapes=[pltpu.VMEM((RA, NP), jnp.float32)],
    )(xA, xA)


def _proj(x, WP, bP, WQ):
    grid = N // RB

    def kern(x_ref, wp_ref, bp_ref, wq_ref, p_ref, q_ref):
        xb = x_ref[...]
        p_ref[...] = jnp.dot(xb, wp_ref[...], preferred_element_type=jnp.float32) + bp_ref[...]
        q_ref[...] = jnp.dot(xb, wq_ref[...], preferred_element_type=jnp.float32)

    return pl.pallas_call(
        kern,
        grid=(grid,),
        in_specs=[
            pl.BlockSpec((RB, D), lambda i: (i, 0)),
            pl.BlockSpec((D, 2 * H), lambda i: (0, 0)),
            pl.BlockSpec((1, 2 * H), lambda i: (0, 0)),
            pl.BlockSpec((D, 2 * H), lambda i: (0, 0)),
        ],
        out_specs=[
            pl.BlockSpec((RB, 2 * H), lambda i: (i, 0)),
            pl.BlockSpec((RB, 2 * H), lambda i: (i, 0)),
        ],
        out_shape=[
            jax.ShapeDtypeStruct((N, 2 * H), jnp.float32),
            jax.ShapeDtypeStruct((N, 2 * H), jnp.float32),
        ],
    )(x, WP, bP, WQ)


def _sc_gather(Qcat, idx_flat):
    vector_mesh = plsc.VectorSubcoreMesh(
        core_axis_name="core", subcore_axis_name="subcore"
    )

    @functools.partial(
        pl.kernel,
        out_type=jax.ShapeDtypeStruct((K * N, 2 * H), jnp.float32),
        mesh=vector_mesh,
    )
    def kern(q_hbm, i_hbm, o_hbm):
        def body(i_vmem, o_vmem):
            pltpu.sync_copy(q_hbm.at[i_vmem.at[0]], o_vmem)

        pltpu.emit_pipeline(
            body,
            grid=(K * N // GW,),
            in_specs=[pl.BlockSpec((1, GW), lambda i: (0, i))],
            out_specs=[pl.BlockSpec((GW, 2 * H), lambda i: (i, 0))],
            core_axis_name=("core", "subcore"),
            dimension_semantics=(pltpu.PARALLEL,),
        )(i_hbm, o_hbm)

    return kern(Qcat, idx_flat)


def _edge_stats(Eq3, Pcat):
    grid = N // RC

    def kern(e_ref, p_ref, o_ref, acc):
        step = pl.program_id(0)

        @pl.when(step == 0)
        def _():
            acc[...] = jnp.zeros((8, 2 * H), jnp.float32)

        e = e_ref[...]
        p = p_ref[...]
        G = jnp.sum(e, axis=0)
        acc[0, :] += jnp.sum(G, axis=0)
        acc[1, :] += jnp.sum(jnp.sum(e * e, axis=0), axis=0)
        acc[2, :] += jnp.sum(p * G, axis=0)
        acc[3, :] += jnp.sum(p, axis=0)
        acc[4, :] += jnp.sum(p * p, axis=0)
        o_ref[0, :] = jnp.float32(K) * acc[3, :] + acc[0, :]
        o_ref[1, :] = jnp.float32(K) * acc[4, :] + 2.0 * acc[2, :] + acc[1, :]

    return pl.pallas_call(
        kern,
        grid=(grid,),
        in_specs=[
            pl.BlockSpec((K, RC, 2 * H), lambda i: (0, i, 0)),
            pl.BlockSpec((RC, 2 * H), lambda i: (i, 0)),
        ],
        out_specs=pl.BlockSpec((8, 2 * H), lambda i: (0, 0)),
        out_shape=jax.ShapeDtypeStruct((8, 2 * H), jnp.float32),
        scratch_shapes=[pltpu.VMEM((8, 2 * H), jnp.float32)],
    )(Eq3, Pcat)


def _edge_mlp(Eq3, Pcat, S, W2s, W2t, b2c, g1c, be1c):
    grid = N // RC
    inv = 1.0 / (N * K)

    def kern(e_ref, p_ref, s_ref, w2s_ref, w2t_ref, b2_ref, g1_ref, be1_ref,
             mx_ref, t_ref, tacc):
        step = pl.program_id(0)

        @pl.when(step == 0)
        def _():
            tacc[...] = jnp.zeros((8, 2 * H), jnp.float32)

        mu1 = s_ref[0, :] * inv
        var1 = s_ref[1, :] * inv - mu1 * mu1
        sc1 = jax.lax.rsqrt(var1 + EPS) * g1_ref[0, :]
        off1 = be1_ref[0, :] - mu1 * sc1
        p = p_ref[...]
        w2s = w2s_ref[...]
        w2t = w2t_ref[...]
        b2 = b2_ref[0, :]
        mx = jnp.full((RC, 2 * H), -jnp.inf, jnp.float32)
        zs = jnp.zeros((2 * H,), jnp.float32)
        zq = jnp.zeros((2 * H,), jnp.float32)
        for k in range(K):
            h = e_ref[k] + p
            r = jnp.maximum(h * sc1 + off1, 0.0)
            z_s = jnp.dot(r[:, :H], w2s, preferred_element_type=jnp.float32)
            z_t = jnp.dot(r[:, H:], w2t, preferred_element_type=jnp.float32)
            z = jnp.concatenate([z_s, z_t], axis=1) + b2
            zs = zs + jnp.sum(z, axis=0)
            zq = zq + jnp.sum(z * z, axis=0)
            mx = jnp.maximum(mx, z)
        mx_ref[...] = mx
        tacc[0, :] += zs
        tacc[1, :] += zq
        t_ref[0, :] = tacc[0, :]
        t_ref[1, :] = tacc[1, :]

    return pl.pallas_call(
        kern,
        grid=(grid,),
        in_specs=[
            pl.BlockSpec((K, RC, 2 * H), lambda i: (0, i, 0)),
            pl.BlockSpec((RC, 2 * H), lambda i: (i, 0)),
            pl.BlockSpec((8, 2 * H), lambda i: (0, 0)),
            pl.BlockSpec((H, H), lambda i: (0, 0)),
            pl.BlockSpec((H, H), lambda i: (0, 0)),
            pl.BlockSpec((1, 2 * H), lambda i: (0, 0)),
            pl.BlockSpec((1, 2 * H), lambda i: (0, 0)),
            pl.BlockSpec((1, 2 * H), lambda i: (0, 0)),
        ],
        out_specs=[
            pl.BlockSpec((RC, 2 * H), lambda i: (i, 0)),
            pl.BlockSpec((8, 2 * H), lambda i: (0, 0)),
        ],
        out_shape=[
            jax.ShapeDtypeStruct((N, 2 * H), jnp.float32),
            jax.ShapeDtypeStruct((8, 2 * H), jnp.float32),
        ],
        scratch_shapes=[pltpu.VMEM((8, 2 * H), jnp.float32)],
    )(Eq3, Pcat, S, W2s, W2t, b2c, g1c, be1c)


def _final_mm(maxcat, T, fW, fbc, g2c, be2c):
    grid = N // RB
    inv = 1.0 / (N * K)

    def kern(mx_ref, t_ref, fw_ref, fb_ref, g2_ref, be2_ref, z3_ref, u_ref, uacc):
        step = pl.program_id(0)

        @pl.when(step == 0)
        def _():
            uacc[...] = jnp.zeros((8, H), jnp.float32)

        mu2 = t_ref[0, :] * inv
        var2 = t_ref[1, :] * inv - mu2 * mu2
        sc2 = jax.lax.rsqrt(var2 + EPS) * g2_ref[0, :]
        off2 = be2_ref[0, :] - mu2 * sc2
        u = jnp.maximum(mx_ref[...] * sc2 + off2, 0.0)
        z3 = jnp.dot(u, fw_ref[...], preferred_element_type=jnp.float32) + fb_ref[0, :]
        z3_ref[...] = z3
        uacc[0, :] += jnp.sum(z3, axis=0)
        uacc[1, :] += jnp.sum(z3 * z3, axis=0)
        u_ref[0, :] = uacc[0, :]
        u_ref[1, :] = uacc[1, :]

    return pl.pallas_call(
        kern,
        grid=(grid,),
        in_specs=[
            pl.BlockSpec((RB, 2 * H), lambda i: (i, 0)),
            pl.BlockSpec((8, 2 * H), lambda i: (0, 0)),
            pl.BlockSpec((2 * H, H), lambda i: (0, 0)),
            pl.BlockSpec((1, H), lambda i: (0, 0)),
            pl.BlockSpec((1, 2 * H), lambda i: (0, 0)),
            pl.BlockSpec((1, 2 * H), lambda i: (0, 0)),
        ],
        out_specs=[
            pl.BlockSpec((RB, H), lambda i: (i, 0)),
            pl.BlockSpec((8, H), lambda i: (0, 0)),
        ],
        out_shape=[
            jax.ShapeDtypeStruct((N, H), jnp.float32),
            jax.ShapeDtypeStruct((8, H), jnp.float32),
        ],
        scratch_shapes=[pltpu.VMEM((8, H), jnp.float32)],
    )(maxcat, T, fW, fbc, g2c, be2c)


def _final_bn(z3, U, fgc, fbec):
    grid = N // RB

    def kern(z3_ref, u_ref, g_ref, be_ref, o_ref):
        mu3 = u_ref[0, :] * (1.0 / N)
        var3 = u_ref[1, :] * (1.0 / N) - mu3 * mu3
        sc3 = jax.lax.rsqrt(var3 + EPS) * g_ref[0, :]
        off3 = be_ref[0, :] - mu3 * sc3
        o_ref[...] = jnp.maximum(z3_ref[...] * sc3 + off3, 0.0)

    return pl.pallas_call(
        kern,
        grid=(grid,),
        in_specs=[
            pl.BlockSpec((RB, H), lambda i: (i, 0)),
            pl.BlockSpec((8, H), lambda i: (0, 0)),
            pl.BlockSpec((1, H), lambda i: (0, 0)),
            pl.BlockSpec((1, H), lambda i: (0, 0)),
        ],
        out_specs=pl.BlockSpec((RB, H), lambda i: (i, 0)),
        out_shape=jax.ShapeDtypeStruct((N, H), jnp.float32),
    )(z3, U, fgc, fbec)


def kernel(x, batch, sW1, sb1, sg1, sbe1, sW2, sb2, sg2, sbe2,
           tW1, tb1, tg1, tbe1, tW2, tb2, tg2, tbe2, fW, fb, fg, fbe):
    xA = jnp.pad(x, ((0, NA - N), (0, 0)))
    WP = jnp.concatenate([sW1[:D] - sW1[D:], tW1[:D] - tW1[D:]], axis=1)
    bP = jnp.concatenate([sb1, tb1])[None, :]
    WQ = jnp.concatenate([sW1[D:], tW1[D:]], axis=1)
    g1c = jnp.concatenate([sg1, tg1])[None, :]
    be1c = jnp.concatenate([sbe1, tbe1])[None, :]
    b2c = jnp.concatenate([sb2, tb2])[None, :]
    g2c = jnp.concatenate([sg2, tg2])[None, :]
    be2c = jnp.concatenate([sbe2, tbe2])[None, :]
    fbc = fb[None, :]
    fgc = fg[None, :]
    fbec = fbe[None, :]

    idxT = _knn(xA)
    idx_flat = idxT[:, :N].reshape(1, K * N)
    Pcat, Qcat = _proj(x, WP, bP, WQ)
    Eq = _sc_gather(Qcat, idx_flat)
    Eq3 = Eq.reshape(K, N, 2 * H)
    S = _edge_stats(Eq3, Pcat)
    maxcat, T = _edge_mlp(Eq3, Pcat, S, sW2, tW2, b2c, g1c, be1c)
    z3, U = _final_mm(maxcat, T, fW, fbc, g2c, be2c)
    return _final_bn(z3, U, fgc, fbec)


# speedup vs baseline: 1.6600x; 1.6600x over previous
import functools

import jax
import jax.numpy as jnp
from jax.experimental import pallas as pl
from jax.experimental.pallas import tpu as pltpu
from jax.experimental.pallas import tpu_sc as plsc

N = 10000
NHA = 5120
NHB = 4880
RAB = 5120
D = 128
H = 128
K = 16
NP = 10112
RA = 512
RB = 400
GW = 128
EPS = 1e-5


def _knn(xT, xRows):
    rows = xRows.shape[0]
    grid = rows // RA

    def kern(xT_ref, xb_ref, idx_ref, strip_ref):
        xt = xT_ref[...]
        sqc = jnp.sum(xt * xt, axis=0, keepdims=True)
        xb = xb_ref[...]
        sqb = jnp.sum(xb * xb, axis=1, keepdims=True)
        dot = jnp.dot(xb, xt, preferred_element_type=jnp.float32)
        cols = jax.lax.broadcasted_iota(jnp.int32, (RA, NP), 1)
        pad = jnp.where(cols >= N, jnp.float32(1e30), jnp.float32(0.0))
        strip_ref[...] = sqb + sqc + pad - 2.0 * dot
        for t in range(K):
            am = jnp.argmin(strip_ref[...], axis=1).astype(jnp.int32)
            idx_ref[t, :] = am
            strip_ref[...] = jnp.where(cols == am[:, None], jnp.float32(1e30),
                                       strip_ref[...])

    return pl.pallas_call(
        kern,
        grid=(grid,),
        in_specs=[
            pl.BlockSpec((D, NP), lambda i: (0, 0)),
            pl.BlockSpec((RA, D), lambda i: (i, 0)),
        ],
        out_specs=pl.BlockSpec((K, RA), lambda i: (0, i)),
        out_shape=jax.ShapeDtypeStruct((K, rows), jnp.int32),
        scratch_shapes=[pltpu.VMEM((RA, NP), jnp.float32)],
    )(xT, xRows)


def _proj(x, WP, bP, WQ):
    grid = N // RB

    def kern(x_ref, wp_ref, bp_ref, wq_ref, p_ref, q_ref):
        xb = x_ref[...]
        p_ref[...] = jnp.dot(xb, wp_ref[...], preferred_element_type=jnp.float32) + bp_ref[...]
        q_ref[...] = jnp.dot(xb, wq_ref[...], preferred_element_type=jnp.float32)

    return pl.pallas_call(
        kern,
        grid=(grid,),
        in_specs=[
            pl.BlockSpec((RB, D), lambda i: (i, 0)),
            pl.BlockSpec((D, 2 * H), lambda i: (0, 0)),
            pl.BlockSpec((1, 2 * H), lambda i: (0, 0)),
            pl.BlockSpec((D, 2 * H), lambda i: (0, 0)),
        ],
        out_specs=[
            pl.BlockSpec((RB, 2 * H), lambda i: (i, 0)),
            pl.BlockSpec((RB, 2 * H), lambda i: (i, 0)),
        ],
        out_shape=[
            jax.ShapeDtypeStruct((N, 2 * H), jnp.float32),
            jax.ShapeDtypeStruct((N, 2 * H), jnp.float32),
        ],
    )(x, WP, bP, WQ)


def _sc_gather(Qcat, idx_flat):
    n_idx = idx_flat.shape[1]
    vector_mesh = plsc.VectorSubcoreMesh(
        core_axis_name="core", subcore_axis_name="subcore"
    )

    @functools.partial(
        pl.kernel,
        out_type=jax.ShapeDtypeStruct((n_idx, 2 * H), jnp.float32),
        mesh=vector_mesh,
    )
    def kern(q_hbm, i_hbm, o_hbm):
        def body(i_vmem, o_vmem):
            pltpu.sync_copy(q_hbm.at[i_vmem.at[0]], o_vmem)

        pltpu.emit_pipeline(
            body,
            grid=(n_idx // GW,),
            in_specs=[pl.BlockSpec((1, GW), lambda i: (0, i))],
            out_specs=[pl.BlockSpec((GW, 2 * H), lambda i: (i, 0))],
            core_axis_name=("core", "subcore"),
            dimension_semantics=(pltpu.PARALLEL,),
        )(i_hbm, o_hbm)

    return kern(Qcat, idx_flat)


def _edge_stats(Eq3, Pslice):
    nh = Pslice.shape[0]
    RC = nh // 10
    grid = 10

    def kern(e_ref, p_ref, o_ref, acc):
        step = pl.program_id(0)

        @pl.when(step == 0)
        def _():
            acc[...] = jnp.zeros((8, 2 * H), jnp.float32)

        e = e_ref[...]
        p = p_ref[...]
        G = jnp.sum(e, axis=0)
        acc[0, :] += jnp.sum(G, axis=0)
        acc[1, :] += jnp.sum(jnp.sum(e * e, axis=0), axis=0)
        acc[2, :] += jnp.sum(p * G, axis=0)
        acc[3, :] += jnp.sum(p, axis=0)
        acc[4, :] += jnp.sum(p * p, axis=0)
        o_ref[0, :] = jnp.float32(K) * acc[3, :] + acc[0, :]
        o_ref[1, :] = jnp.float32(K) * acc[4, :] + 2.0 * acc[2, :] + acc[1, :]

    return pl.pallas_call(
        kern,
        grid=(grid,),
        in_specs=[
            pl.BlockSpec((K, RC, 2 * H), lambda i: (0, i, 0)),
            pl.BlockSpec((RC, 2 * H), lambda i: (i, 0)),
        ],
        out_specs=pl.BlockSpec((8, 2 * H), lambda i: (0, 0)),
        out_shape=jax.ShapeDtypeStruct((8, 2 * H), jnp.float32),
        scratch_shapes=[pltpu.VMEM((8, 2 * H), jnp.float32)],
    )(Eq3, Pslice)


def _edge_mlp(Eq3, Pslice, S, W2s, W2t, b2c, g1c, be1c):
    nh = Pslice.shape[0]
    RC = nh // 10
    grid = 10
    inv = 1.0 / (N * K)

    def kern(e_ref, p_ref, s_ref, w2s_ref, w2t_ref, b2_ref, g1_ref, be1_ref,
             mx_ref, t_ref, tacc):
        step = pl.program_id(0)

        @pl.when(step == 0)
        def _():
            tacc[...] = jnp.zeros((8, 2 * H), jnp.float32)

        mu1 = s_ref[0, :] * inv
        var1 = s_ref[1, :] * inv - mu1 * mu1
        sc1 = jax.lax.rsqrt(var1 + EPS) * g1_ref[0, :]
        off1 = be1_ref[0, :] - mu1 * sc1
        p = p_ref[...]
        w2s = w2s_ref[...]
        w2t = w2t_ref[...]
        b2 = b2_ref[0, :]
        mx = jnp.full((RC, 2 * H), -jnp.inf, jnp.float32)
        zs = jnp.zeros((2 * H,), jnp.float32)
        zq = jnp.zeros((2 * H,), jnp.float32)
        for k in range(K):
            h = e_ref[k] + p
            r = jnp.maximum(h * sc1 + off1, 0.0)
            z_s = jnp.dot(r[:, :H], w2s, preferred_element_type=jnp.float32)
            z_t = jnp.dot(r[:, H:], w2t, preferred_element_type=jnp.float32)
            z = jnp.concatenate([z_s, z_t], axis=1) + b2
            zs = zs + jnp.sum(z, axis=0)
            zq = zq + jnp.sum(z * z, axis=0)
            mx = jnp.maximum(mx, z)
        mx_ref[...] = mx
        tacc[0, :] += zs
        tacc[1, :] += zq
        t_ref[0, :] = tacc[0, :]
        t_ref[1, :] = tacc[1, :]

    return pl.pallas_call(
        kern,
        grid=(grid,),
        in_specs=[
            pl.BlockSpec((K, RC, 2 * H), lambda i: (0, i, 0)),
            pl.BlockSpec((RC, 2 * H), lambda i: (i, 0)),
            pl.BlockSpec((8, 2 * H), lambda i: (0, 0)),
            pl.BlockSpec((H, H), lambda i: (0, 0)),
            pl.BlockSpec((H, H), lambda i: (0, 0)),
            pl.BlockSpec((1, 2 * H), lambda i: (0, 0)),
            pl.BlockSpec((1, 2 * H), lambda i: (0, 0)),
            pl.BlockSpec((1, 2 * H), lambda i: (0, 0)),
        ],
        out_specs=[
            pl.BlockSpec((RC, 2 * H), lambda i: (i, 0)),
            pl.BlockSpec((8, 2 * H), lambda i: (0, 0)),
        ],
        out_shape=[
            jax.ShapeDtypeStruct((nh, 2 * H), jnp.float32),
            jax.ShapeDtypeStruct((8, 2 * H), jnp.float32),
        ],
        scratch_shapes=[pltpu.VMEM((8, 2 * H), jnp.float32)],
    )(Eq3, Pslice, S, W2s, W2t, b2c, g1c, be1c)


def _final_mm(maxcat, T, fW, fbc, g2c, be2c):
    grid = N // RB
    inv = 1.0 / (N * K)

    def kern(mx_ref, t_ref, fw_ref, fb_ref, g2_ref, be2_ref, z3_ref, u_ref, uacc):
        step = pl.program_id(0)

        @pl.when(step == 0)
        def _():
            uacc[...] = jnp.zeros((8, H), jnp.float32)

        mu2 = t_ref[0, :] * inv
        var2 = t_ref[1, :] * inv - mu2 * mu2
        sc2 = jax.lax.rsqrt(var2 + EPS) * g2_ref[0, :]
        off2 = be2_ref[0, :] - mu2 * sc2
        u = jnp.maximum(mx_ref[...] * sc2 + off2, 0.0)
        z3 = jnp.dot(u, fw_ref[...], preferred_element_type=jnp.float32) + fb_ref[0, :]
        z3_ref[...] = z3
        uacc[0, :] += jnp.sum(z3, axis=0)
        uacc[1, :] += jnp.sum(z3 * z3, axis=0)
        u_ref[0, :] = uacc[0, :]
        u_ref[1, :] = uacc[1, :]

    return pl.pallas_call(
        kern,
        grid=(grid,),
        in_specs=[
            pl.BlockSpec((RB, 2 * H), lambda i: (i, 0)),
            pl.BlockSpec((8, 2 * H), lambda i: (0, 0)),
            pl.BlockSpec((2 * H, H), lambda i: (0, 0)),
            pl.BlockSpec((1, H), lambda i: (0, 0)),
            pl.BlockSpec((1, 2 * H), lambda i: (0, 0)),
            pl.BlockSpec((1, 2 * H), lambda i: (0, 0)),
        ],
        out_specs=[
            pl.BlockSpec((RB, H), lambda i: (i, 0)),
            pl.BlockSpec((8, H), lambda i: (0, 0)),
        ],
        out_shape=[
            jax.ShapeDtypeStruct((N, H), jnp.float32),
            jax.ShapeDtypeStruct((8, H), jnp.float32),
        ],
        scratch_shapes=[pltpu.VMEM((8, H), jnp.float32)],
    )(maxcat, T, fW, fbc, g2c, be2c)


def _final_bn(z3, U, fgc, fbec):
    grid = N // RB

    def kern(z3_ref, u_ref, g_ref, be_ref, o_ref):
        mu3 = u_ref[0, :] * (1.0 / N)
        var3 = u_ref[1, :] * (1.0 / N) - mu3 * mu3
        sc3 = jax.lax.rsqrt(var3 + EPS) * g_ref[0, :]
        off3 = be_ref[0, :] - mu3 * sc3
        o_ref[...] = jnp.maximum(z3_ref[...] * sc3 + off3, 0.0)

    return pl.pallas_call(
        kern,
        grid=(grid,),
        in_specs=[
            pl.BlockSpec((RB, H), lambda i: (i, 0)),
            pl.BlockSpec((8, H), lambda i: (0, 0)),
            pl.BlockSpec((1, H), lambda i: (0, 0)),
            pl.BlockSpec((1, H), lambda i: (0, 0)),
        ],
        out_specs=pl.BlockSpec((RB, H), lambda i: (i, 0)),
        out_shape=jax.ShapeDtypeStruct((N, H), jnp.float32),
    )(z3, U, fgc, fbec)


def kernel(x, batch, sW1, sb1, sg1, sbe1, sW2, sb2, sg2, sbe2,
           tW1, tb1, tg1, tbe1, tW2, tb2, tg2, tbe2, fW, fb, fg, fbe):
    xT = jnp.pad(x, ((0, NP - N), (0, 0))).T
    WP = jnp.concatenate([sW1[:D] - sW1[D:], tW1[:D] - tW1[D:]], axis=1)
    bP = jnp.concatenate([sb1, tb1])[None, :]
    WQ = jnp.concatenate([sW1[D:], tW1[D:]], axis=1)
    g1c = jnp.concatenate([sg1, tg1])[None, :]
    be1c = jnp.concatenate([sbe1, tbe1])[None, :]
    b2c = jnp.concatenate([sb2, tb2])[None, :]
    g2c = jnp.concatenate([sg2, tg2])[None, :]
    be2c = jnp.concatenate([sbe2, tbe2])[None, :]
    fbc = fb[None, :]
    fgc = fg[None, :]
    fbec = fbe[None, :]

    Pcat, Qcat = _proj(x, WP, bP, WQ)
    idxT_a = _knn(xT, x[:NHA])
    Eq_a = _sc_gather(Qcat, idxT_a.reshape(1, K * NHA))
    xb_pad = jnp.pad(x[NHA:], ((0, RAB - NHB), (0, 0)))
    idxT_b = _knn(xT, xb_pad)[:, :NHB]
    Eq_b = _sc_gather(Qcat, idxT_b.reshape(1, K * NHB))
    Pa, Pb = Pcat[:NHA], Pcat[NHA:]
    Sa = _edge_stats(Eq_a.reshape(K, NHA, 2 * H), Pa)
    Sb = _edge_stats(Eq_b.reshape(K, NHB, 2 * H), Pb)
    S = Sa + Sb
    mx_a, Ta = _edge_mlp(Eq_a.reshape(K, NHA, 2 * H), Pa, S, sW2, tW2, b2c, g1c, be1c)
    mx_b, Tb = _edge_mlp(Eq_b.reshape(K, NHB, 2 * H), Pb, S, sW2, tW2, b2c, g1c, be1c)
    maxcat = jnp.concatenate([mx_a, mx_b], axis=0)
    T = Ta + Tb
    z3, U = _final_mm(maxcat, T, fW, fbc, g2c, be2c)
    return _final_bn(z3, U, fgc, fbec)


# speedup vs baseline: 1.6673x; 1.0044x over previous
import functools

import jax
import jax.numpy as jnp
from jax.experimental import pallas as pl
from jax.experimental.pallas import tpu as pltpu
from jax.experimental.pallas import tpu_sc as plsc

N = 10000
NHA = 5120
NHB = 4880
RAB = 5120
D = 128
H = 128
K = 16
NP = 10112
RA = 512
RB = 400
GW = 128
EPS = 1e-5


def _knn(xT, xRows):
    rows = xRows.shape[0]
    grid = rows // RA

    def kern(xT_ref, xb_ref, idx_ref, strip_ref):
        xt = xT_ref[...]
        sqc = jnp.sum(xt * xt, axis=0, keepdims=True)
        xb = xb_ref[...]
        sqb = jnp.sum(xb * xb, axis=1, keepdims=True)
        dot = jnp.dot(xb, xt, preferred_element_type=jnp.float32)
        cols = jax.lax.broadcasted_iota(jnp.int32, (RA, NP), 1)
        pad = jnp.where(cols >= N, jnp.float32(1e30), jnp.float32(0.0))
        strip_ref[...] = sqb + sqc + pad - 2.0 * dot
        for g in range(K // 2):
            s = strip_ref[...]
            am1 = jnp.argmin(s, axis=1).astype(jnp.int32)
            s2 = jnp.where(cols == am1[:, None], jnp.float32(1e30), s)
            am2 = jnp.argmin(s2, axis=1).astype(jnp.int32)
            idx_ref[2 * g, :] = am1
            idx_ref[2 * g + 1, :] = am2
            strip_ref[...] = jnp.where(cols == am2[:, None], jnp.float32(1e30),
                                       s2)

    return pl.pallas_call(
        kern,
        grid=(grid,),
        in_specs=[
            pl.BlockSpec((D, NP), lambda i: (0, 0)),
            pl.BlockSpec((RA, D), lambda i: (i, 0)),
        ],
        out_specs=pl.BlockSpec((K, RA), lambda i: (0, i)),
        out_shape=jax.ShapeDtypeStruct((K, rows), jnp.int32),
        scratch_shapes=[pltpu.VMEM((RA, NP), jnp.float32)],
    )(xT, xRows)


def _proj(x, WP, bP, WQ):
    grid = N // RB

    def kern(x_ref, wp_ref, bp_ref, wq_ref, p_ref, q_ref):
        xb = x_ref[...]
        p_ref[...] = jnp.dot(xb, wp_ref[...], preferred_element_type=jnp.float32) + bp_ref[...]
        q_ref[...] = jnp.dot(xb, wq_ref[...], preferred_element_type=jnp.float32)

    return pl.pallas_call(
        kern,
        grid=(grid,),
        in_specs=[
            pl.BlockSpec((RB, D), lambda i: (i, 0)),
            pl.BlockSpec((D, 2 * H), lambda i: (0, 0)),
            pl.BlockSpec((1, 2 * H), lambda i: (0, 0)),
            pl.BlockSpec((D, 2 * H), lambda i: (0, 0)),
        ],
        out_specs=[
            pl.BlockSpec((RB, 2 * H), lambda i: (i, 0)),
            pl.BlockSpec((RB, 2 * H), lambda i: (i, 0)),
        ],
        out_shape=[
            jax.ShapeDtypeStruct((N, 2 * H), jnp.float32),
            jax.ShapeDtypeStruct((N, 2 * H), jnp.float32),
        ],
    )(x, WP, bP, WQ)


def _sc_gather(Qcat, idx_flat):
    n_idx = idx_flat.shape[1]
    vector_mesh = plsc.VectorSubcoreMesh(
        core_axis_name="core", subcore_axis_name="subcore"
    )

    @functools.partial(
        pl.kernel,
        out_type=jax.ShapeDtypeStruct((n_idx, 2 * H), jnp.float32),
        mesh=vector_mesh,
    )
    def kern(q_hbm, i_hbm, o_hbm):
        def body(i_vmem, o_vmem):
            pltpu.sync_copy(q_hbm.at[i_vmem.at[0]], o_vmem)

        pltpu.emit_pipeline(
            body,
            grid=(n_idx // GW,),
            in_specs=[pl.BlockSpec((1, GW), lambda i: (0, i))],
            out_specs=[pl.BlockSpec((GW, 2 * H), lambda i: (i, 0))],
            core_axis_name=("core", "subcore"),
            dimension_semantics=(pltpu.PARALLEL,),
        )(i_hbm, o_hbm)

    return kern(Qcat, idx_flat)


def _edge_stats(Eq3, Pslice):
    nh = Pslice.shape[0]
    RC = nh // 10
    grid = 10

    def kern(e_ref, p_ref, o_ref, acc):
        step = pl.program_id(0)

        @pl.when(step == 0)
        def _():
            acc[...] = jnp.zeros((8, 2 * H), jnp.float32)

        e = e_ref[...]
        p = p_ref[...]
        G = jnp.sum(e, axis=0)
        acc[0, :] += jnp.sum(G, axis=0)
        acc[1, :] += jnp.sum(jnp.sum(e * e, axis=0), axis=0)
        acc[2, :] += jnp.sum(p * G, axis=0)
        acc[3, :] += jnp.sum(p, axis=0)
        acc[4, :] += jnp.sum(p * p, axis=0)
        o_ref[0, :] = jnp.float32(K) * acc[3, :] + acc[0, :]
        o_ref[1, :] = jnp.float32(K) * acc[4, :] + 2.0 * acc[2, :] + acc[1, :]

    return pl.pallas_call(
        kern,
        grid=(grid,),
        in_specs=[
            pl.BlockSpec((K, RC, 2 * H), lambda i: (0, i, 0)),
            pl.BlockSpec((RC, 2 * H), lambda i: (i, 0)),
        ],
        out_specs=pl.BlockSpec((8, 2 * H), lambda i: (0, 0)),
        out_shape=jax.ShapeDtypeStruct((8, 2 * H), jnp.float32),
        scratch_shapes=[pltpu.VMEM((8, 2 * H), jnp.float32)],
    )(Eq3, Pslice)


def _edge_mlp(Eq3, Pslice, S, W2s, W2t, b2c, g1c, be1c):
    nh = Pslice.shape[0]
    RC = nh // 10
    grid = 10
    inv = 1.0 / (N * K)

    def kern(e_ref, p_ref, s_ref, w2s_ref, w2t_ref, b2_ref, g1_ref, be1_ref,
             mx_ref, t_ref, tacc):
        step = pl.program_id(0)

        @pl.when(step == 0)
        def _():
            tacc[...] = jnp.zeros((8, 2 * H), jnp.float32)

        mu1 = s_ref[0, :] * inv
        var1 = s_ref[1, :] * inv - mu1 * mu1
        sc1 = jax.lax.rsqrt(var1 + EPS) * g1_ref[0, :]
        off1 = be1_ref[0, :] - mu1 * sc1
        p = p_ref[...]
        w2s = w2s_ref[...]
        w2t = w2t_ref[...]
        b2 = b2_ref[0, :]
        mx = jnp.full((RC, 2 * H), -jnp.inf, jnp.float32)
        zs = jnp.zeros((2 * H,), jnp.float32)
        zq = jnp.zeros((2 * H,), jnp.float32)
        for k in range(K):
            h = e_ref[k] + p
            r = jnp.maximum(h * sc1 + off1, 0.0)
            z_s = jnp.dot(r[:, :H], w2s, preferred_element_type=jnp.float32)
            z_t = jnp.dot(r[:, H:], w2t, preferred_element_type=jnp.float32)
            z = jnp.concatenate([z_s, z_t], axis=1) + b2
            zs = zs + jnp.sum(z, axis=0)
            zq = zq + jnp.sum(z * z, axis=0)
            mx = jnp.maximum(mx, z)
        mx_ref[...] = mx
        tacc[0, :] += zs
        tacc[1, :] += zq
        t_ref[0, :] = tacc[0, :]
        t_ref[1, :] = tacc[1, :]

    return pl.pallas_call(
        kern,
        grid=(grid,),
        in_specs=[
            pl.BlockSpec((K, RC, 2 * H), lambda i: (0, i, 0)),
            pl.BlockSpec((RC, 2 * H), lambda i: (i, 0)),
            pl.BlockSpec((8, 2 * H), lambda i: (0, 0)),
            pl.BlockSpec((H, H), lambda i: (0, 0)),
            pl.BlockSpec((H, H), lambda i: (0, 0)),
            pl.BlockSpec((1, 2 * H), lambda i: (0, 0)),
            pl.BlockSpec((1, 2 * H), lambda i: (0, 0)),
            pl.BlockSpec((1, 2 * H), lambda i: (0, 0)),
        ],
        out_specs=[
            pl.BlockSpec((RC, 2 * H), lambda i: (i, 0)),
            pl.BlockSpec((8, 2 * H), lambda i: (0, 0)),
        ],
        out_shape=[
            jax.ShapeDtypeStruct((nh, 2 * H), jnp.float32),
            jax.ShapeDtypeStruct((8, 2 * H), jnp.float32),
        ],
        scratch_shapes=[pltpu.VMEM((8, 2 * H), jnp.float32)],
    )(Eq3, Pslice, S, W2s, W2t, b2c, g1c, be1c)


def _final_mm(maxcat, T, fW, fbc, g2c, be2c):
    grid = N // RB
    inv = 1.0 / (N * K)

    def kern(mx_ref, t_ref, fw_ref, fb_ref, g2_ref, be2_ref, z3_ref, u_ref, uacc):
        step = pl.program_id(0)

        @pl.when(step == 0)
        def _():
            uacc[...] = jnp.zeros((8, H), jnp.float32)

        mu2 = t_ref[0, :] * inv
        var2 = t_ref[1, :] * inv - mu2 * mu2
        sc2 = jax.lax.rsqrt(var2 + EPS) * g2_ref[0, :]
        off2 = be2_ref[0, :] - mu2 * sc2
        u = jnp.maximum(mx_ref[...] * sc2 + off2, 0.0)
        z3 = jnp.dot(u, fw_ref[...], preferred_element_type=jnp.float32) + fb_ref[0, :]
        z3_ref[...] = z3
        uacc[0, :] += jnp.sum(z3, axis=0)
        uacc[1, :] += jnp.sum(z3 * z3, axis=0)
        u_ref[0, :] = uacc[0, :]
        u_ref[1, :] = uacc[1, :]

    return pl.pallas_call(
        kern,
        grid=(grid,),
        in_specs=[
            pl.BlockSpec((RB, 2 * H), lambda i: (i, 0)),
            pl.BlockSpec((8, 2 * H), lambda i: (0, 0)),
            pl.BlockSpec((2 * H, H), lambda i: (0, 0)),
            pl.BlockSpec((1, H), lambda i: (0, 0)),
            pl.BlockSpec((1, 2 * H), lambda i: (0, 0)),
            pl.BlockSpec((1, 2 * H), lambda i: (0, 0)),
        ],
        out_specs=[
            pl.BlockSpec((RB, H), lambda i: (i, 0)),
            pl.BlockSpec((8, H), lambda i: (0, 0)),
        ],
        out_shape=[
            jax.ShapeDtypeStruct((N, H), jnp.float32),
            jax.ShapeDtypeStruct((8, H), jnp.float32),
        ],
        scratch_shapes=[pltpu.VMEM((8, H), jnp.float32)],
    )(maxcat, T, fW, fbc, g2c, be2c)


def _final_bn(z3, U, fgc, fbec):
    grid = N // RB

    def kern(z3_ref, u_ref, g_ref, be_ref, o_ref):
        mu3 = u_ref[0, :] * (1.0 / N)
        var3 = u_ref[1, :] * (1.0 / N) - mu3 * mu3
        sc3 = jax.lax.rsqrt(var3 + EPS) * g_ref[0, :]
        off3 = be_ref[0, :] - mu3 * sc3
        o_ref[...] = jnp.maximum(z3_ref[...] * sc3 + off3, 0.0)

    return pl.pallas_call(
        kern,
        grid=(grid,),
        in_specs=[
            pl.BlockSpec((RB, H), lambda i: (i, 0)),
            pl.BlockSpec((8, H), lambda i: (0, 0)),
            pl.BlockSpec((1, H), lambda i: (0, 0)),
            pl.BlockSpec((1, H), lambda i: (0, 0)),
        ],
        out_specs=pl.BlockSpec((RB, H), lambda i: (i, 0)),
        out_shape=jax.ShapeDtypeStruct((N, H), jnp.float32),
    )(z3, U, fgc, fbec)


def kernel(x, batch, sW1, sb1, sg1, sbe1, sW2, sb2, sg2, sbe2,
           tW1, tb1, tg1, tbe1, tW2, tb2, tg2, tbe2, fW, fb, fg, fbe):
    xT = jnp.pad(x, ((0, NP - N), (0, 0))).T
    WP = jnp.concatenate([sW1[:D] - sW1[D:], tW1[:D] - tW1[D:]], axis=1)
    bP = jnp.concatenate([sb1, tb1])[None, :]
    WQ = jnp.concatenate([sW1[D:], tW1[D:]], axis=1)
    g1c = jnp.concatenate([sg1, tg1])[None, :]
    be1c = jnp.concatenate([sbe1, tbe1])[None, :]
    b2c = jnp.concatenate([sb2, tb2])[None, :]
    g2c = jnp.concatenate([sg2, tg2])[None, :]
    be2c = jnp.concatenate([sbe2, tbe2])[None, :]
    fbc = fb[None, :]
    fgc = fg[None, :]
    fbec = fbe[None, :]

    Pcat, Qcat = _proj(x, WP, bP, WQ)
    idxT_a = _knn(xT, x[:NHA])
    Eq_a = _sc_gather(Qcat, idxT_a.reshape(1, K * NHA))
    xb_pad = jnp.pad(x[NHA:], ((0, RAB - NHB), (0, 0)))
    idxT_b = _knn(xT, xb_pad)[:, :NHB]
    Eq_b = _sc_gather(Qcat, idxT_b.reshape(1, K * NHB))
    Pa, Pb = Pcat[:NHA], Pcat[NHA:]
    Sa = _edge_stats(Eq_a.reshape(K, NHA, 2 * H), Pa)
    Sb = _edge_stats(Eq_b.reshape(K, NHB, 2 * H), Pb)
    S = Sa + Sb
    mx_a, Ta = _edge_mlp(Eq_a.reshape(K, NHA, 2 * H), Pa, S, sW2, tW2, b2c, g1c, be1c)
    mx_b, Tb = _edge_mlp(Eq_b.reshape(K, NHB, 2 * H), Pb, S, sW2, tW2, b2c, g1c, be1c)
    maxcat = jnp.concatenate([mx_a, mx_b], axis=0)
    T = Ta + Tb
    z3, U = _final_mm(maxcat, T, fW, fbc, g2c, be2c)
    return _final_bn(z3, U, fgc, fbec)
